# baseline (speedup 1.0000x reference)
import functools

import numpy as np
import jax
import jax.numpy as jnp
from jax import lax
from jax.experimental import pallas as pl
from jax.experimental.pallas import tpu as pltpu
from jax.experimental.pallas import tpu_sc as plsc

VOCAB_N = 100000
EMB_N = 128
SEQ_N = 200
BATCH_N = 1024

NUM_CORES = 2
NUM_SUBCORES = 16
NUM_WORKERS = NUM_CORES * NUM_SUBCORES

CHUNK = 160
SEQ_PER_W = BATCH_N // NUM_WORKERS
IDX_PER_W = SEQ_PER_W * SEQ_N
CHUNKS_PER_W = IDX_PER_W // CHUNK
LANES = 16
NBUF = 4


def _pe_table() -> np.ndarray:
    pos = np.arange(SEQ_N, dtype=np.float32)[:, None]
    i = np.arange(EMB_N, dtype=np.float32)[None, :]
    angle = pos / np.power(np.float32(1000.0), np.float32(2.0) * i / np.float32(EMB_N))
    pe = np.where((np.arange(EMB_N)[None, :] % 2) == 0, np.sin(angle), np.cos(angle))
    return pe.astype(np.float32)


_mesh = plsc.VectorSubcoreMesh(
    core_axis_name="c", subcore_axis_name="s",
    num_cores=NUM_CORES, num_subcores=NUM_SUBCORES,
)


@functools.partial(
    pl.kernel,
    out_type=jax.ShapeDtypeStruct((BATCH_N * SEQ_N, EMB_N), jnp.float32),
    mesh=_mesh,
    scratch_types=[
        pltpu.VMEM((SEQ_PER_W * SEQ_N,), jnp.int32),
        pltpu.VMEM((NBUF, CHUNK, EMB_N), jnp.float32),
        pltpu.VMEM((SEQ_N * EMB_N,), jnp.float32),
        [pltpu.SemaphoreType.DMA] * NBUF,
        [pltpu.SemaphoreType.DMA] * NBUF,
    ],
)
def _emb_kernel(x_hbm, table_hbm, pe_hbm, out_hbm, idx_v, rows_v, pe_v, gsem, osem):
    wid = lax.axis_index("s") * NUM_CORES + lax.axis_index("c")
    idx_base = wid * SEQ_PER_W * SEQ_N
    pltpu.sync_copy(
        x_hbm.at[pl.ds(pl.multiple_of(idx_base, 8), SEQ_PER_W * SEQ_N)], idx_v)

    def idx_slice(j):
        return idx_v.at[pl.ds(pl.multiple_of(j * CHUNK, 8), CHUNK)]

    def start_gather(j, b):
        return pltpu.async_copy(table_hbm.at[idx_slice(j)], rows_v.at[b], gsem[b])

    def wait_gather(b):
        pltpu.make_async_copy(
            table_hbm.at[idx_slice(0)], rows_v.at[b], gsem[b]).wait()

    out_base = wid * SEQ_PER_W * SEQ_N

    def out_slice(j):
        row0 = out_base + j * CHUNK
        return out_hbm.at[pl.ds(pl.multiple_of(row0, 8), CHUNK)]

    def wait_out(b):
        pltpu.make_async_copy(rows_v.at[b], out_slice(0), osem[b]).wait()

    def step_fn(j, b, wait_prev_out):
        wait_gather(b)

        @plsc.parallel_loop(0, CHUNK, step=1, unroll=4)
        def _(i, _b=b, _j=j):
            p = lax.rem(_j * CHUNK + i, SEQ_N)
            pe_off = pl.multiple_of(p * EMB_N, LANES)
            for c in range(EMB_N // LANES):
                plsc.addupdate(
                    rows_v.at[_b, i, pl.ds(c * LANES, LANES)],
                    pe_v[pl.ds(pe_off + c * LANES, LANES)],
                )

        pltpu.async_copy(rows_v.at[b], out_slice(j), osem[b])
        bn = (b + NBUF - 1) % NBUF
        if wait_prev_out:
            wait_out(bn)
        jn = j + (NBUF - 1)
        if isinstance(jn, int):
            if jn < CHUNKS_PER_W:
                start_gather(jn, bn)
        else:
            @pl.when(jn < CHUNKS_PER_W)
            def _():
                start_gather(jn, bn)

    for j in range(NBUF - 1):
        start_gather(j, j)
    pltpu.sync_copy(pe_hbm, pe_v)
    for j in range(NBUF):
        step_fn(j, j, wait_prev_out=(j > 0))

    @pl.loop(NBUF, CHUNKS_PER_W, step=NBUF)
    def _(j0):
        for b in range(NBUF):
            step_fn(j0 + b, b, wait_prev_out=True)

    wait_out((CHUNKS_PER_W - 1) % NBUF)


def kernel(x, table):
    pe = jnp.asarray(_pe_table())
    out_flat = _emb_kernel(x.reshape(BATCH_N * SEQ_N), table, pe.reshape(SEQ_N * EMB_N))
    return out_flat.reshape(BATCH_N, SEQ_N, EMB_N)


# speedup vs baseline: 1.0090x; 1.0090x over previous
import functools

import numpy as np
import jax
import jax.numpy as jnp
from jax import lax
from jax.experimental import pallas as pl
from jax.experimental.pallas import tpu as pltpu
from jax.experimental.pallas import tpu_sc as plsc

VOCAB_N = 100000
EMB_N = 128
SEQ_N = 200
BATCH_N = 1024

NUM_CORES = 2
NUM_SUBCORES = 16
NUM_WORKERS = NUM_CORES * NUM_SUBCORES

CHUNK = 128
SEQ_PER_W = BATCH_N // NUM_WORKERS
IDX_PER_W = SEQ_PER_W * SEQ_N
CHUNKS_PER_W = IDX_PER_W // CHUNK
LANES = 16
NBUF = 5


def _pe_table() -> np.ndarray:
    pos = np.arange(SEQ_N, dtype=np.float32)[:, None]
    i = np.arange(EMB_N, dtype=np.float32)[None, :]
    angle = pos / np.power(np.float32(1000.0), np.float32(2.0) * i / np.float32(EMB_N))
    pe = np.where((np.arange(EMB_N)[None, :] % 2) == 0, np.sin(angle), np.cos(angle))
    return pe.astype(np.float32)


_mesh = plsc.VectorSubcoreMesh(
    core_axis_name="c", subcore_axis_name="s",
    num_cores=NUM_CORES, num_subcores=NUM_SUBCORES,
)


@functools.partial(
    pl.kernel,
    out_type=jax.ShapeDtypeStruct((BATCH_N * SEQ_N, EMB_N), jnp.float32),
    mesh=_mesh,
    scratch_types=[
        pltpu.VMEM((SEQ_PER_W * SEQ_N,), jnp.int32),
        pltpu.VMEM((NBUF, CHUNK, EMB_N), jnp.float32),
        pltpu.VMEM((SEQ_N * EMB_N,), jnp.float32),
        [pltpu.SemaphoreType.DMA] * NBUF,
        [pltpu.SemaphoreType.DMA] * NBUF,
    ],
)
def _emb_kernel(x_hbm, table_hbm, pe_hbm, out_hbm, idx_v, rows_v, pe_v, gsem, osem):
    wid = lax.axis_index("s") * NUM_CORES + lax.axis_index("c")
    idx_base = wid * SEQ_PER_W * SEQ_N
    pltpu.sync_copy(
        x_hbm.at[pl.ds(pl.multiple_of(idx_base, 8), SEQ_PER_W * SEQ_N)], idx_v)

    def idx_slice(j):
        return idx_v.at[pl.ds(pl.multiple_of(j * CHUNK, 8), CHUNK)]

    def start_gather(j, b):
        return pltpu.async_copy(table_hbm.at[idx_slice(j)], rows_v.at[b], gsem[b])

    def wait_gather(b):
        pltpu.make_async_copy(
            table_hbm.at[idx_slice(0)], rows_v.at[b], gsem[b]).wait()

    out_base = wid * SEQ_PER_W * SEQ_N

    def out_slice(j):
        row0 = out_base + j * CHUNK
        return out_hbm.at[pl.ds(pl.multiple_of(row0, 8), CHUNK)]

    def wait_out(b):
        pltpu.make_async_copy(rows_v.at[b], out_slice(0), osem[b]).wait()

    def step_fn(j, b, wait_prev_out):
        wait_gather(b)

        @plsc.parallel_loop(0, CHUNK, step=1, unroll=4)
        def _(i, _b=b, _j=j):
            p = lax.rem(_j * CHUNK + i, SEQ_N)
            pe_off = pl.multiple_of(p * EMB_N, LANES)
            for c in range(EMB_N // LANES):
                plsc.addupdate(
                    rows_v.at[_b, i, pl.ds(c * LANES, LANES)],
                    pe_v[pl.ds(pe_off + c * LANES, LANES)],
                )

        pltpu.async_copy(rows_v.at[b], out_slice(j), osem[b])
        bn = (b + NBUF - 1) % NBUF
        if wait_prev_out:
            wait_out(bn)
        jn = j + (NBUF - 1)
        if isinstance(jn, int):
            if jn < CHUNKS_PER_W:
                start_gather(jn, bn)
        else:
            @pl.when(jn < CHUNKS_PER_W)
            def _():
                start_gather(jn, bn)

    for j in range(NBUF - 1):
        start_gather(j, j)
    pltpu.sync_copy(pe_hbm, pe_v)
    for j in range(NBUF):
        step_fn(j, j, wait_prev_out=(j > 0))

    @pl.loop(NBUF, CHUNKS_PER_W, step=NBUF)
    def _(j0):
        for b in range(NBUF):
            step_fn(j0 + b, b, wait_prev_out=True)

    wait_out((CHUNKS_PER_W - 1) % NBUF)


def kernel(x, table):
    pe = jnp.asarray(_pe_table())
    out_flat = _emb_kernel(x.reshape(BATCH_N * SEQ_N), table, pe.reshape(SEQ_N * EMB_N))
    return out_flat.reshape(BATCH_N, SEQ_N, EMB_N)
